# baseline (speedup 1.0000x reference)
import jax
import jax.numpy as jnp
from jax.experimental import pallas as pl

TOPK = 16
BB = 8
NEG_INF = float("-inf")


def _consensus_body(lite_ref, in_ref, out_ref):
    scores = jnp.max(lite_ref[...], axis=2)
    t_iota = jax.lax.broadcasted_iota(jnp.int32, scores.shape, 1)
    selected = jnp.zeros(scores.shape, jnp.bool_)
    big = jnp.int32(2**30)
    for _ in range(TOPK):
        m = jnp.max(scores, axis=1, keepdims=True)
        cand = jnp.where(scores == m, t_iota, big)
        idx = jnp.min(cand, axis=1, keepdims=True)
        hit = t_iota == idx
        selected = jnp.logical_or(selected, hit)
        scores = jnp.where(hit, NEG_INF, scores)
    w = jnp.where(selected, 1.0 / TOPK, 0.0)
    acc = jax.lax.dot_general(
        w,
        in_ref[...],
        dimension_numbers=(((1,), (1,)), ((0,), (0,))),
        preferred_element_type=jnp.float32,
    )
    out_ref[:, 0, :] = acc


@jax.jit
def kernel(input, lite_input):
    B, T, D = input.shape

    out = pl.pallas_call(
        _consensus_body,
        grid=(B // BB,),
        in_specs=[
            pl.BlockSpec((BB, T, D), lambda b: (b, 0, 0)),
            pl.BlockSpec((BB, T, D), lambda b: (b, 0, 0)),
        ],
        out_specs=pl.BlockSpec((BB, 1, D), lambda b: (b, 0, 0)),
        out_shape=jax.ShapeDtypeStruct((B, 1, D), jnp.float32),
    )(lite_input, input)

    return out


# speedup vs baseline: 1.0494x; 1.0494x over previous
import jax
import jax.numpy as jnp
from jax import lax
from jax.experimental import pallas as pl
from jax.experimental.pallas import tpu as pltpu
from jax.experimental.pallas import tpu_sc as plsc

TOPK = 16
LANES = 16
NEG_INF = float("-inf")
BB = 16
BPW = 2
SPAD = 128


def _scores_body(lite_ref, scores_ref):
    s = jnp.max(lite_ref[...], axis=2)
    pad = jnp.zeros((BB, SPAD - s.shape[1]), jnp.float32)
    scores_ref[...] = jnp.concatenate([s, pad], axis=1)


def _xor_reduce(v, op):
    iota = lax.broadcasted_iota(jnp.int32, (LANES,), 0)
    for s in (8, 4, 2, 1):
        v = op(v, v.at[iota ^ s].get(mode="promise_in_bounds"))
    return v


def _sc_topk_gather_mean_body(
    scores_hbm, in_hbm, out_hbm, sc_v, rows_v, idx_v, out_v, gsem0, gsem1
):
    D = in_hbm.shape[1]
    T = 64
    nc = 2
    wid = lax.axis_index("s") * nc + lax.axis_index("c")
    iota = lax.broadcasted_iota(jnp.int32, (LANES,), 0)
    pltpu.sync_copy(scores_hbm.at[pl.ds(wid * BPW, BPW)], sc_v)

    gsems = [gsem0, gsem1]
    big = jnp.int32(2**30)
    for bb in range(BPW):
        svecs = [sc_v[bb, pl.ds(j * LANES, LANES)] for j in range(4)]
        idx_acc = jnp.zeros((LANES,), jnp.int32)
        for k in range(TOPK):
            m = jnp.maximum(
                jnp.maximum(svecs[0], svecs[1]), jnp.maximum(svecs[2], svecs[3])
            )
            mx = _xor_reduce(m, jnp.maximum)
            cands = [
                jnp.where(svecs[j] == mx, iota + j * LANES, big) for j in range(4)
            ]
            cmin = jnp.minimum(
                jnp.minimum(cands[0], cands[1]), jnp.minimum(cands[2], cands[3])
            )
            t = _xor_reduce(cmin, jnp.minimum)
            idx_acc = jnp.where(iota == k, (wid * BPW + bb) * T + t, idx_acc)
            for j in range(4):
                svecs[j] = jnp.where(iota + j * LANES == t, NEG_INF, svecs[j])
        idx_v[bb, :] = idx_acc
        pltpu.make_async_copy(
            in_hbm.at[idx_v.at[bb]], rows_v.at[bb], gsems[bb]
        ).start()

    for bb in range(BPW):
        pltpu.make_async_copy(
            in_hbm.at[idx_v.at[bb]], rows_v.at[bb], gsems[bb]
        ).wait()

        @pl.loop(0, D // (2 * LANES))
        def _mean(cidx):
            for h in range(2):
                sl = pl.ds(cidx * 2 * LANES + h * LANES, LANES)
                vals = [rows_v[bb, r, sl] for r in range(TOPK)]
                while len(vals) > 1:
                    vals = [vals[i] + vals[i + 1] for i in range(0, len(vals), 2)]
                out_v[bb, 0, sl] = vals[0] * (1.0 / TOPK)

    pltpu.sync_copy(out_v, out_hbm.at[pl.ds(wid * BPW, BPW)])


@jax.jit
def kernel(input, lite_input):
    B, T, D = input.shape

    scores = pl.pallas_call(
        _scores_body,
        grid=(B // BB,),
        in_specs=[pl.BlockSpec((BB, T, D), lambda b: (b, 0, 0))],
        out_specs=pl.BlockSpec((BB, SPAD), lambda b: (b, 0)),
        out_shape=jax.ShapeDtypeStruct((B, SPAD), jnp.float32),
    )(lite_input)

    input_rows = input.reshape(B * T, D)

    sc_stage = pl.kernel(
        _sc_topk_gather_mean_body,
        out_type=jax.ShapeDtypeStruct((B, 1, D), jnp.float32),
        mesh=plsc.VectorSubcoreMesh(core_axis_name="c", subcore_axis_name="s"),
        scratch_types=[
            pltpu.VMEM((BPW, SPAD), jnp.float32),
            pltpu.VMEM((BPW, TOPK, D), jnp.float32),
            pltpu.VMEM((BPW, TOPK), jnp.int32),
            pltpu.VMEM((BPW, 1, D), jnp.float32),
            pltpu.SemaphoreType.DMA,
            pltpu.SemaphoreType.DMA,
        ],
    )
    return sc_stage(scores, input_rows)
